# baseline (speedup 1.0000x reference)
import functools

import jax
import jax.numpy as jnp
from jax import lax
from jax.experimental import pallas as pl
from jax.experimental.pallas import tpu as pltpu
from jax.experimental.pallas import tpu_sc as plsc


def _sc_geometry():
    try:
        info = plsc.get_sparse_core_info()
        return info.num_cores, info.num_subcores, info.num_lanes
    except Exception:
        return 2, 16, 16


@functools.cache
def _build(B: int, D: int, A_pad: int):
    NC, NS, L = _sc_geometry()
    NW = NC * NS
    assert B % (8 * NW) == 0, "batch must split 8-aligned across subcores"
    b_per_w = B // NW
    mesh = plsc.VectorSubcoreMesh(core_axis_name="c", subcore_axis_name="s")

    @functools.partial(
        pl.kernel,
        mesh=mesh,
        out_type=jax.ShapeDtypeStruct((B, D), jnp.float32),
        scratch_types=[
            pltpu.VMEM((b_per_w,), jnp.int32),
            pltpu.VMEM((b_per_w,), jnp.int32),
            pltpu.VMEM((b_per_w,), jnp.int32),
            pltpu.VMEM((b_per_w,), jnp.int32),
            pltpu.VMEM((b_per_w, D), jnp.float32),
            pltpu.SemaphoreType.DMA,
        ],
        compiler_params=pltpu.CompilerParams(use_tc_tiling_on_sc=False),
    )
    def k(a_hbm, s_hbm, off_hbm, table_hbm, out_hbm,
          aidx_v, sidx_v, offg_v, idx_v, rows_v, sem):
        wid = lax.axis_index("s") * NC + lax.axis_index("c")
        base = wid * b_per_w
        pltpu.sync_copy(a_hbm.at[pl.ds(base, b_per_w)], aidx_v)
        pltpu.sync_copy(s_hbm.at[pl.ds(base, b_per_w)], sidx_v)
        pltpu.async_copy(off_hbm.at[aidx_v], offg_v, sem).wait()
        for i in range(b_per_w // L):
            idx_v[pl.ds(i * L, L)] = (offg_v[pl.ds(i * L, L)]
                                      + sidx_v[pl.ds(i * L, L)])
        pltpu.async_copy(table_hbm.at[idx_v], rows_v, sem).wait()
        pltpu.sync_copy(rows_v, out_hbm.at[pl.ds(base, b_per_w)])

    return k


def kernel(asset_index, shape_index, sub_embedding_sizes, offsets, table):
    del sub_embedding_sizes
    B = asset_index.shape[0]
    D = table.shape[1]
    A = offsets.shape[0]
    A_pad = -(-A // 128) * 128
    offs_p = jnp.zeros((A_pad,), jnp.int32).at[:A].set(offsets)
    return _build(B, D, A_pad)(asset_index, shape_index, offs_p, table)


# speedup vs baseline: 1.2229x; 1.2229x over previous
import functools

import jax
import jax.numpy as jnp
from jax import lax
from jax.experimental import pallas as pl
from jax.experimental.pallas import tpu as pltpu
from jax.experimental.pallas import tpu_sc as plsc


def _sc_geometry():
    try:
        info = plsc.get_sparse_core_info()
        return info.num_cores, info.num_subcores, info.num_lanes
    except Exception:
        return 2, 16, 16


@functools.cache
def _build(B: int, D: int, A: int):
    NC, NS, L = _sc_geometry()
    NW = NC * NS
    assert B % (8 * NW) == 0, "batch must split 8-aligned across subcores"
    b_per_w = B // NW
    mesh = plsc.VectorSubcoreMesh(core_axis_name="c", subcore_axis_name="s")
    A_pad = -(-A // L) * L

    @functools.partial(
        pl.kernel,
        mesh=mesh,
        out_type=jax.ShapeDtypeStruct((B, D), jnp.float32),
        scratch_types=[
            pltpu.VMEM((b_per_w,), jnp.int32),
            pltpu.VMEM((b_per_w,), jnp.int32),
            pltpu.VMEM((b_per_w,), jnp.int32),
            pltpu.VMEM((A_pad,), jnp.int32),
            pltpu.VMEM((b_per_w, D), jnp.float32),
            pltpu.SemaphoreType.DMA,
            pltpu.SemaphoreType.DMA,
            pltpu.SemaphoreType.DMA,
        ],
        compiler_params=pltpu.CompilerParams(
            use_tc_tiling_on_sc=False, needs_layout_passes=False),
    )
    def k(a_hbm, s_hbm, off_hbm, table_hbm, out_hbm,
          aidx_v, sidx_v, idx_v, offs_v, rows_v, sem_a, sem_s, sem_o):
        wid = lax.axis_index("s") * NC + lax.axis_index("c")
        base = wid * b_per_w
        cp_a = pltpu.make_async_copy(a_hbm.at[pl.ds(base, b_per_w)], aidx_v, sem_a)
        cp_s = pltpu.make_async_copy(s_hbm.at[pl.ds(base, b_per_w)], sidx_v, sem_s)
        cp_o = pltpu.make_async_copy(off_hbm, offs_v.at[pl.ds(0, A)], sem_o)
        cp_a.start(); cp_s.start(); cp_o.start()
        cp_a.wait(); cp_s.wait(); cp_o.wait()
        for i in range(b_per_w // L):
            a = aidx_v[pl.ds(i * L, L)]
            off = plsc.load_gather(offs_v, [a])
            idx_v[pl.ds(i * L, L)] = off + sidx_v[pl.ds(i * L, L)]
        pltpu.async_copy(table_hbm.at[idx_v], rows_v, sem_a).wait()
        pltpu.sync_copy(rows_v, out_hbm.at[pl.ds(base, b_per_w)])

    return k


def kernel(asset_index, shape_index, sub_embedding_sizes, offsets, table):
    del sub_embedding_sizes
    B = asset_index.shape[0]
    D = table.shape[1]
    A = offsets.shape[0]
    return _build(B, D, A)(asset_index, shape_index, offsets, table)


# speedup vs baseline: 1.7055x; 1.3946x over previous
import functools

import jax
import jax.numpy as jnp
from jax import lax
from jax.experimental import pallas as pl
from jax.experimental.pallas import tpu as pltpu
from jax.experimental.pallas import tpu_sc as plsc


def _sc_geometry():
    try:
        info = plsc.get_sparse_core_info()
        return info.num_cores, info.num_subcores, info.num_lanes
    except Exception:
        return 2, 16, 16


@functools.cache
def _build(B: int, T: int, D: int, A: int):
    NC, NS, L = _sc_geometry()
    NW = NC * NS
    assert B % (8 * NW) == 0, "batch must split 8-aligned across subcores"
    b_per_w = B // NW
    BATCH = 32
    n_batches = b_per_w // BATCH
    mesh = plsc.VectorSubcoreMesh(core_axis_name="c", subcore_axis_name="s")

    @functools.partial(
        pl.kernel,
        mesh=mesh,
        out_type=jax.ShapeDtypeStruct((B, D), jnp.float32),
        scratch_types=[
            pltpu.VMEM((b_per_w,), jnp.int32),
            pltpu.VMEM((b_per_w,), jnp.int32),
            pltpu.VMEM((b_per_w,), jnp.int32),
            pltpu.VMEM((-(-A // L) * L,), jnp.int32),
            pltpu.VMEM((BATCH, 8, D), jnp.float32),
            pltpu.VMEM((BATCH, 8, D), jnp.float32),
            pltpu.VMEM((b_per_w, D), jnp.float32),
            pltpu.SemaphoreType.DMA,
            pltpu.SemaphoreType.DMA,
        ],
        compiler_params=pltpu.CompilerParams(
            use_tc_tiling_on_sc=True, needs_layout_passes=False),
    )
    def k(a_hbm, s_hbm, off_hbm, tab_hbm, out_hbm,
          a_v, s_v, idx_v, off_v, buf_a, buf_b, rows_v, sem0, sem1):
        wid = lax.axis_index("s") * NC + lax.axis_index("c")
        base = wid * b_per_w
        pltpu.sync_copy(a_hbm.at[pl.ds(base, b_per_w)], a_v)
        pltpu.sync_copy(s_hbm.at[pl.ds(base, b_per_w)], s_v)
        pltpu.sync_copy(off_hbm, off_v.at[pl.ds(0, A)])
        for i in range(b_per_w // L):
            a = a_v[pl.ds(i * L, L)]
            off = plsc.load_gather(off_v, [a])
            idx_v[pl.ds(i * L, L)] = off + s_v[pl.ds(i * L, L)]

        bufs = (buf_a, buf_b)
        sems = (sem0, sem1)

        def fire(batch):
            buf, sem = bufs[batch % 2], sems[batch % 2]
            descs = []
            for g in range(BATCH // L):
                vec = idx_v[pl.ds(batch * BATCH + g * L, L)]
                for lane in range(L):
                    c = vec[lane]
                    descs.append(pltpu.make_async_copy(
                        tab_hbm.at[pl.ds(c >> 3, 1)],
                        buf.at[pl.ds(g * L + lane, 1)], sem))
            for d in descs:
                d.start()
            return descs

        def extract(batch):
            buf = bufs[batch % 2]
            for g in range(BATCH // L):
                vec = idx_v[pl.ds(batch * BATCH + g * L, L)]
                for lane in range(L):
                    jj = g * L + lane
                    j = batch * BATCH + jj
                    r = vec[lane] & 7
                    for kk in range(D // L):
                        rows_v[j, pl.ds(kk * L, L)] = buf[jj, r, pl.ds(kk * L, L)]

        prev = fire(0)
        for batch in range(1, n_batches):
            cur = fire(batch)
            for d in prev:
                d.wait()
            extract(batch - 1)
            prev = cur
        for d in prev:
            d.wait()
        extract(n_batches - 1)
        pltpu.sync_copy(rows_v, out_hbm.at[pl.ds(base, b_per_w)])

    return k


def kernel(asset_index, shape_index, sub_embedding_sizes, offsets, table):
    del sub_embedding_sizes
    B = asset_index.shape[0]
    V, D = table.shape
    A = offsets.shape[0]
    assert V % 8 == 0
    tab3 = table.reshape(V // 8, 8, D)
    return _build(B, V // 8, D, A)(asset_index, shape_index, offsets, tab3)


# speedup vs baseline: 1.7250x; 1.0114x over previous
import functools

import jax
import jax.numpy as jnp
from jax import lax
from jax.experimental import pallas as pl
from jax.experimental.pallas import tpu as pltpu
from jax.experimental.pallas import tpu_sc as plsc


def _sc_geometry():
    try:
        info = plsc.get_sparse_core_info()
        return info.num_cores, info.num_subcores, info.num_lanes
    except Exception:
        return 2, 16, 16


@functools.cache
def _build(B: int, T: int, D: int, A: int):
    NC, NS, L = _sc_geometry()
    NW = NC * NS
    assert B % (8 * NW) == 0, "batch must split 8-aligned across subcores"
    b_per_w = B // NW
    BATCH = 32
    n_batches = b_per_w // BATCH
    mesh = plsc.VectorSubcoreMesh(core_axis_name="c", subcore_axis_name="s")

    @functools.partial(
        pl.kernel,
        mesh=mesh,
        out_type=jax.ShapeDtypeStruct((D, B), jnp.float32),
        scratch_types=[
            pltpu.VMEM((b_per_w,), jnp.int32),
            pltpu.VMEM((b_per_w,), jnp.int32),
            pltpu.VMEM((b_per_w,), jnp.int32),
            pltpu.VMEM((-(-A // L) * L,), jnp.int32),
            pltpu.VMEM((BATCH, 8, D), jnp.float32),
            pltpu.VMEM((BATCH, 8, D), jnp.float32),
            pltpu.VMEM((D, b_per_w), jnp.float32),
            pltpu.SemaphoreType.DMA,
            pltpu.SemaphoreType.DMA,
        ],
        compiler_params=pltpu.CompilerParams(
            use_tc_tiling_on_sc=True, needs_layout_passes=False),
    )
    def k(a_hbm, s_hbm, off_hbm, tab_hbm, out_hbm,
          a_v, s_v, idx_v, off_v, buf_a, buf_b, rows_v, sem0, sem1):
        wid = lax.axis_index("s") * NC + lax.axis_index("c")
        base = wid * b_per_w
        pltpu.sync_copy(a_hbm.at[pl.ds(base, b_per_w)], a_v)
        pltpu.sync_copy(s_hbm.at[pl.ds(base, b_per_w)], s_v)
        pltpu.sync_copy(off_hbm, off_v.at[pl.ds(0, A)])
        for i in range(b_per_w // L):
            a = a_v[pl.ds(i * L, L)]
            off = plsc.load_gather(off_v, [a])
            idx_v[pl.ds(i * L, L)] = off + s_v[pl.ds(i * L, L)]

        bufs = (buf_a, buf_b)
        sems = (sem0, sem1)

        def fire(batch):
            buf, sem = bufs[batch % 2], sems[batch % 2]
            descs = []
            for g in range(BATCH // L):
                vec = idx_v[pl.ds(batch * BATCH + g * L, L)]
                for lane in range(L):
                    c = vec[lane]
                    descs.append(pltpu.make_async_copy(
                        tab_hbm.at[pl.ds(c >> 3, 1)],
                        buf.at[pl.ds(g * L + lane, 1)], sem))
            for d in descs:
                d.start()
            return descs

        row_ids = [lax.iota(jnp.int32, L) + kk * L for kk in range(D // L)]

        def extract(batch):
            buf = bufs[batch % 2]
            for g in range(BATCH // L):
                vec = idx_v[pl.ds(batch * BATCH + g * L, L)]
                for lane in range(L):
                    jj = g * L + lane
                    j = batch * BATCH + jj
                    r = vec[lane] & 7
                    col = jnp.full((L,), j, jnp.int32)
                    for kk in range(D // L):
                        x = buf[jj, r, pl.ds(kk * L, L)]
                        plsc.store_scatter(rows_v, [row_ids[kk], col], x)

        prev = fire(0)
        for batch in range(1, n_batches):
            cur = fire(batch)
            for d in prev:
                d.wait()
            extract(batch - 1)
            prev = cur
        for d in prev:
            d.wait()
        extract(n_batches - 1)
        pltpu.sync_copy(rows_v, out_hbm.at[:, pl.ds(base, b_per_w)])

    return k


def kernel(asset_index, shape_index, sub_embedding_sizes, offsets, table):
    del sub_embedding_sizes
    B = asset_index.shape[0]
    V, D = table.shape
    A = offsets.shape[0]
    assert V % 8 == 0
    tab3 = table.reshape(V // 8, 8, D)
    out_t = _build(B, V // 8, D, A)(asset_index, shape_index, offsets, tab3)
    return out_t.T


# speedup vs baseline: 1.8528x; 1.0741x over previous
import functools

import jax
import jax.numpy as jnp
from jax import lax
from jax.experimental import pallas as pl
from jax.experimental.pallas import tpu as pltpu
from jax.experimental.pallas import tpu_sc as plsc


def _sc_geometry():
    try:
        info = plsc.get_sparse_core_info()
        return info.num_cores, info.num_subcores, info.num_lanes
    except Exception:
        return 2, 16, 16


@functools.cache
def _build(B: int, T: int, D: int, A: int):
    NC, NS, L = _sc_geometry()
    NW = NC * NS
    assert B % (8 * NW) == 0, "batch must split 8-aligned across subcores"
    b_per_w = B // NW
    BATCH = 32
    n_batches = b_per_w // BATCH
    mesh = plsc.VectorSubcoreMesh(core_axis_name="c", subcore_axis_name="s")

    @functools.partial(
        pl.kernel,
        mesh=mesh,
        out_type=jax.ShapeDtypeStruct((D, B), jnp.float32),
        scratch_types=[
            pltpu.VMEM((b_per_w,), jnp.int32),
            pltpu.VMEM((b_per_w,), jnp.int32),
            pltpu.VMEM((b_per_w,), jnp.int32),
            pltpu.VMEM((-(-A // L) * L,), jnp.int32),
            pltpu.VMEM((BATCH, 8, D), jnp.float32),
            pltpu.VMEM((BATCH, 8, D), jnp.float32),
            pltpu.VMEM((D, b_per_w), jnp.float32),
            pltpu.SemaphoreType.DMA,
            pltpu.SemaphoreType.DMA,
            pltpu.SemaphoreType.DMA,
        ],
        compiler_params=pltpu.CompilerParams(
            use_tc_tiling_on_sc=True, needs_layout_passes=False),
    )
    def k(a_hbm, s_hbm, off_hbm, tab_hbm, out_hbm,
          a_v, s_v, idx_v, off_v, buf_a, buf_b, rows_v, sem0, sem1, sem2):
        wid = lax.axis_index("s") * NC + lax.axis_index("c")
        base = wid * b_per_w
        cp_a = pltpu.make_async_copy(a_hbm.at[pl.ds(base, b_per_w)], a_v, sem0)
        cp_s = pltpu.make_async_copy(s_hbm.at[pl.ds(base, b_per_w)], s_v, sem1)
        cp_o = pltpu.make_async_copy(off_hbm, off_v.at[pl.ds(0, A)], sem2)
        cp_a.start(); cp_s.start(); cp_o.start()
        cp_a.wait(); cp_s.wait(); cp_o.wait()
        for i in range(b_per_w // L):
            a = a_v[pl.ds(i * L, L)]
            off = plsc.load_gather(off_v, [a])
            idx_v[pl.ds(i * L, L)] = off + s_v[pl.ds(i * L, L)]

        bufs = (buf_a, buf_b)
        sems = (sem0, sem1)
        row_ids = [lax.iota(jnp.int32, L) + kk * L for kk in range(D // L)]

        def fire(batch):
            buf, sem = bufs[batch % 2], sems[batch % 2]

            def body(g, carry):
                vec = idx_v[pl.ds(batch * BATCH + g * L, L)]
                for lane in range(L):
                    tid = vec[lane] >> 3
                    pltpu.make_async_copy(
                        tab_hbm.at[pl.ds(tid, 1)],
                        buf.at[pl.ds(g * L + lane, 1)], sem).start()
                return carry

            lax.fori_loop(0, BATCH // L, body, 0)

        def drain(batch):
            buf, sem = bufs[batch % 2], sems[batch % 2]
            pltpu.make_async_copy(tab_hbm.at[pl.ds(0, BATCH)], buf, sem).wait()

        def extract(batch):
            buf = bufs[batch % 2]

            def body(g, carry):
                j0 = batch * BATCH + g * L
                vec = idx_v[pl.ds(j0, L)]
                for lane in range(L):
                    r = vec[lane] & 7
                    jj = g * L + lane
                    col = jnp.full((L,), j0 + lane, jnp.int32)
                    for kk in range(D // L):
                        x = buf[jj, r, pl.ds(kk * L, L)]
                        plsc.store_scatter(rows_v, [row_ids[kk], col], x)
                return carry

            lax.fori_loop(0, BATCH // L, body, 0)

        fire(0)
        for batch in range(1, n_batches):
            fire(batch)
            drain(batch - 1)
            extract(batch - 1)
        drain(n_batches - 1)
        extract(n_batches - 1)
        pltpu.sync_copy(rows_v, out_hbm.at[:, pl.ds(base, b_per_w)])

    return k


def kernel(asset_index, shape_index, sub_embedding_sizes, offsets, table):
    del sub_embedding_sizes
    B = asset_index.shape[0]
    V, D = table.shape
    A = offsets.shape[0]
    assert V % 8 == 0
    tab3 = table.reshape(V // 8, 8, D)
    out_t = _build(B, V // 8, D, A)(asset_index, shape_index, offsets, tab3)
    return out_t.T


# speedup vs baseline: 1.8645x; 1.0063x over previous
import functools

import jax
import jax.numpy as jnp
from jax import lax
from jax.experimental import pallas as pl
from jax.experimental.pallas import tpu as pltpu
from jax.experimental.pallas import tpu_sc as plsc


def _sc_geometry():
    try:
        info = plsc.get_sparse_core_info()
        return info.num_cores, info.num_subcores, info.num_lanes
    except Exception:
        return 2, 16, 16


@functools.cache
def _build(B: int, T: int, D: int, A: int):
    NC, NS, L = _sc_geometry()
    NW = NC * NS
    assert B % (8 * NW) == 0, "batch must split 8-aligned across subcores"
    b_per_w = B // NW
    BATCH = 32
    n_batches = b_per_w // BATCH
    mesh = plsc.VectorSubcoreMesh(core_axis_name="c", subcore_axis_name="s")

    @functools.partial(
        pl.kernel,
        mesh=mesh,
        out_type=jax.ShapeDtypeStruct((D, B), jnp.float32),
        scratch_types=[
            pltpu.VMEM((b_per_w,), jnp.int32),
            pltpu.VMEM((b_per_w,), jnp.int32),
            pltpu.VMEM((b_per_w,), jnp.int32),
            pltpu.VMEM((-(-A // L) * L,), jnp.int32),
            pltpu.VMEM((BATCH, 8, D), jnp.float32),
            pltpu.VMEM((BATCH, 8, D), jnp.float32),
            pltpu.VMEM((BATCH, 8, D), jnp.float32),
            pltpu.VMEM((D, b_per_w), jnp.float32),
            pltpu.SemaphoreType.DMA,
            pltpu.SemaphoreType.DMA,
            pltpu.SemaphoreType.DMA,
        ],
        compiler_params=pltpu.CompilerParams(
            use_tc_tiling_on_sc=True, needs_layout_passes=False),
    )
    def k(a_hbm, s_hbm, off_hbm, tab_hbm, out_hbm,
          a_v, s_v, idx_v, off_v, buf_a, buf_b, buf_c, rows_v,
          sem0, sem1, sem2):
        wid = lax.axis_index("s") * NC + lax.axis_index("c")
        base = wid * b_per_w
        cp_a = pltpu.make_async_copy(a_hbm.at[pl.ds(base, b_per_w)], a_v, sem0)
        cp_s = pltpu.make_async_copy(s_hbm.at[pl.ds(base, b_per_w)], s_v, sem1)
        cp_o = pltpu.make_async_copy(off_hbm, off_v.at[pl.ds(0, A)], sem2)
        cp_a.start(); cp_s.start(); cp_o.start()
        cp_a.wait(); cp_s.wait(); cp_o.wait()
        for i in range(b_per_w // L):
            a = a_v[pl.ds(i * L, L)]
            off = plsc.load_gather(off_v, [a])
            idx_v[pl.ds(i * L, L)] = off + s_v[pl.ds(i * L, L)]

        bufs = (buf_a, buf_b, buf_c)
        sems = (sem0, sem1, sem2)
        DEPTH = len(bufs)
        row_ids = [lax.iota(jnp.int32, L) + kk * L for kk in range(D // L)]

        def fire(batch):
            buf, sem = bufs[batch % DEPTH], sems[batch % DEPTH]

            def body(g, carry):
                vec = idx_v[pl.ds(batch * BATCH + g * L, L)]
                for lane in range(L):
                    tid = vec[lane] >> 3
                    pltpu.make_async_copy(
                        tab_hbm.at[pl.ds(tid, 1)],
                        buf.at[pl.ds(g * L + lane, 1)], sem).start()
                return carry

            lax.fori_loop(0, BATCH // L, body, 0)

        def drain(batch):
            buf, sem = bufs[batch % DEPTH], sems[batch % DEPTH]
            pltpu.make_async_copy(tab_hbm.at[pl.ds(0, BATCH)], buf, sem).wait()

        def extract(batch):
            buf = bufs[batch % DEPTH]

            def body(g, carry):
                j0 = batch * BATCH + g * L
                vec = idx_v[pl.ds(j0, L)]
                for lane in range(L):
                    r = vec[lane] & 7
                    jj = g * L + lane
                    col = jnp.full((L,), j0 + lane, jnp.int32)
                    for kk in range(D // L):
                        x = buf[jj, r, pl.ds(kk * L, L)]
                        plsc.store_scatter(rows_v, [row_ids[kk], col], x)
                return carry

            lax.fori_loop(0, BATCH // L, body, 0)

        for batch in range(min(DEPTH, n_batches)):
            fire(batch)
        for batch in range(n_batches):
            drain(batch)
            if batch + DEPTH < n_batches:
                fire(batch + DEPTH)
            extract(batch)
        pltpu.sync_copy(rows_v, out_hbm.at[:, pl.ds(base, b_per_w)])

    return k


def kernel(asset_index, shape_index, sub_embedding_sizes, offsets, table):
    del sub_embedding_sizes
    B = asset_index.shape[0]
    V, D = table.shape
    A = offsets.shape[0]
    assert V % 8 == 0
    tab3 = table.reshape(V // 8, 8, D)
    out_t = _build(B, V // 8, D, A)(asset_index, shape_index, offsets, tab3)
    return out_t.T
